# Initial kernel scaffold; baseline (speedup 1.0000x reference)
import functools

import jax
import jax.numpy as jnp
from jax import lax
from jax.experimental import pallas as pl
from jax.experimental.pallas import tpu as pltpu
from jax.experimental.pallas import tpu_sc as plsc

LANES = 16


@functools.lru_cache(maxsize=None)
def _build(batch: int):
    info = plsc.get_sparse_core_info()
    nc, ns = info.num_cores, info.num_subcores
    nw = nc * ns
    assert batch % (8 * nw) == 0, "HBM 1-D slice offsets must be 8-aligned"
    b_per_w = batch // nw

    mesh = plsc.VectorSubcoreMesh(core_axis_name="c", subcore_axis_name="s")

    @functools.partial(
        pl.kernel,
        mesh=mesh,
        out_type=jax.ShapeDtypeStruct((batch,), jnp.float32),
        scratch_types=[
            pltpu.VMEM((b_per_w,), jnp.int32),
            pltpu.VMEM((b_per_w,), jnp.float32),
            pltpu.VMEM((LANES,), jnp.float32),
            pltpu.VMEM((LANES,), jnp.float32),
        ],
    )
    def sc_kernel(idx_hbm, embpat_hbm, wpat_hbm, out_hbm, idx_v, out_v, ep_v, wp_v):
        wid = lax.axis_index("s") * nc + lax.axis_index("c")
        base = wid * b_per_w
        pltpu.sync_copy(idx_hbm.at[pl.ds(base, b_per_w)], idx_v)
        pltpu.sync_copy(embpat_hbm, ep_v)
        pltpu.sync_copy(wpat_hbm, wp_v)

        prod = ep_v[...] * wp_v[...]
        lane = lax.iota(jnp.int32, LANES)
        zero = jnp.zeros((LANES,), jnp.float32)
        bias_lane = lane == 12
        l0 = jnp.sum(jnp.where((lane < 4) | bias_lane, prod, zero))
        l1 = jnp.sum(jnp.where(((lane >= 4) & (lane < 8)) | bias_lane, prod, zero))
        l2 = jnp.sum(jnp.where(((lane >= 8) & (lane < 12)) | bias_lane, prod, zero))
        one = jnp.ones((LANES,), jnp.float32)
        s0 = one / (one + jnp.exp(jnp.broadcast_to(-l0, (LANES,))))
        s1 = one / (one + jnp.exp(jnp.broadcast_to(-l1, (LANES,))))
        s2 = one / (one + jnp.exp(jnp.broadcast_to(-l2, (LANES,))))

        for j in range(b_per_w // LANES):
            iv = idx_v[pl.ds(j * LANES, LANES)]
            out_v[pl.ds(j * LANES, LANES)] = jnp.where(
                iv == 0, s0, jnp.where(iv == 1, s1, s2)
            )

        pltpu.sync_copy(out_v, out_hbm.at[pl.ds(base, b_per_w)])

    return sc_kernel


def kernel(indices, emb_table, W, b):
    emb_pat = jnp.concatenate(
        [emb_table.reshape(-1), b.reshape(-1), jnp.zeros((3,), jnp.float32)]
    )
    w_pat = jnp.concatenate(
        [
            jnp.tile(W.reshape(-1), 3),
            jnp.ones((1,), jnp.float32),
            jnp.zeros((3,), jnp.float32),
        ]
    )
    out = _build(indices.shape[0])(indices.astype(jnp.int32), emb_pat, w_pat)
    return out.reshape(-1, 1)


# speedup vs baseline: 2.1788x; 2.1788x over previous
import functools

import jax
import jax.numpy as jnp
from jax import lax
from jax.experimental import pallas as pl
from jax.experimental.pallas import tpu as pltpu
from jax.experimental.pallas import tpu_sc as plsc

LANES = 16


@functools.lru_cache(maxsize=None)
def _build(batch: int):
    info = plsc.get_sparse_core_info()
    nc, ns = info.num_cores, info.num_subcores
    nw = nc * ns
    assert batch % (8 * nw) == 0, "HBM 1-D slice offsets must be 8-aligned"
    b_per_w = batch // nw

    mesh = plsc.VectorSubcoreMesh(core_axis_name="c", subcore_axis_name="s")

    @functools.partial(
        pl.kernel,
        mesh=mesh,
        out_type=jax.ShapeDtypeStruct((batch,), jnp.float32),
        compiler_params=pltpu.CompilerParams(needs_layout_passes=False),
        scratch_types=[
            pltpu.VMEM((b_per_w,), jnp.int32),
            pltpu.VMEM((b_per_w,), jnp.float32),
            pltpu.VMEM((LANES,), jnp.float32),
            pltpu.VMEM((LANES,), jnp.float32),
            pltpu.VMEM((LANES,), jnp.float32),
        ],
    )
    def sc_kernel(idx_hbm, embpat_hbm, wpat_hbm, out_hbm, idx_v, out_v, ep_v, wp_v, scr_v):
        wid = lax.axis_index("s") * nc + lax.axis_index("c")
        base = wid * b_per_w
        pltpu.sync_copy(idx_hbm.at[pl.ds(base, b_per_w)], idx_v)
        pltpu.sync_copy(embpat_hbm, ep_v)
        pltpu.sync_copy(wpat_hbm, wp_v)

        lane = lax.iota(jnp.int32, LANES)
        scr_v[...] = ep_v[...] * wp_v[...]
        seg = lane & jnp.full((LANES,), -4, jnp.int32)
        one_i = jnp.ones((LANES,), jnp.int32)
        sum4 = (
            plsc.load_gather(scr_v, [seg])
            + plsc.load_gather(scr_v, [seg + one_i])
            + plsc.load_gather(scr_v, [seg + 2 * one_i])
            + plsc.load_gather(scr_v, [seg + 3 * one_i])
        )
        scr_v[...] = sum4
        bias = plsc.load_gather(scr_v, [jnp.full((LANES,), 12, jnp.int32)])
        one = jnp.ones((LANES,), jnp.float32)
        sig = one / (one + jnp.exp(-(sum4 + bias)))
        scr_v[...] = sig
        table_idx = jnp.minimum(lane * 4, jnp.full((LANES,), 12, jnp.int32))
        scr_v[...] = plsc.load_gather(scr_v, [table_idx])

        for j in range(b_per_w // LANES):
            iv = idx_v[pl.ds(j * LANES, LANES)]
            out_v[pl.ds(j * LANES, LANES)] = plsc.load_gather(scr_v, [iv])

        pltpu.sync_copy(out_v, out_hbm.at[pl.ds(base, b_per_w)])

    return sc_kernel


def kernel(indices, emb_table, W, b):
    emb_pat = jnp.concatenate(
        [emb_table.reshape(-1), b.reshape(-1), jnp.zeros((3,), jnp.float32)]
    )
    w_pat = jnp.concatenate(
        [
            jnp.tile(W.reshape(-1), 3),
            jnp.ones((1,), jnp.float32),
            jnp.zeros((3,), jnp.float32),
        ]
    )
    out = _build(indices.shape[0])(indices.astype(jnp.int32), emb_pat, w_pat)
    return out.reshape(-1, 1)


# speedup vs baseline: 2.2805x; 1.0467x over previous
import functools

import jax
import jax.numpy as jnp
from jax import lax
from jax.experimental import pallas as pl
from jax.experimental.pallas import tpu as pltpu
from jax.experimental.pallas import tpu_sc as plsc

LANES = 16


@functools.lru_cache(maxsize=None)
def _build(batch: int):
    info = plsc.get_sparse_core_info()
    nc, ns = info.num_cores, info.num_subcores
    nw = nc * ns
    assert batch % (8 * nw) == 0, "HBM 1-D slice offsets must be 8-aligned"
    b_per_w = batch // nw

    mesh = plsc.VectorSubcoreMesh(core_axis_name="c", subcore_axis_name="s")

    @functools.partial(
        pl.kernel,
        mesh=mesh,
        out_type=jax.ShapeDtypeStruct((batch,), jnp.float32),
        compiler_params=pltpu.CompilerParams(needs_layout_passes=False),
        scratch_types=[
            pltpu.VMEM((b_per_w,), jnp.int32),
            pltpu.VMEM((b_per_w,), jnp.float32),
            pltpu.VMEM((3, 4), jnp.float32),
            pltpu.VMEM((4, 1), jnp.float32),
            pltpu.VMEM((1,), jnp.float32),
            pltpu.VMEM((LANES,), jnp.float32),
            pltpu.SemaphoreType.DMA,
        ],
    )
    def sc_kernel(idx_hbm, emb_hbm, w_hbm, b_hbm, out_hbm, idx_v, out_v, e_v, w_v, b_v, scr_v, sem):
        wid = lax.axis_index("s") * nc + lax.axis_index("c")
        base = wid * b_per_w
        cps = [
            pltpu.async_copy(idx_hbm.at[pl.ds(base, b_per_w)], idx_v, sem),
            pltpu.async_copy(emb_hbm, e_v, sem),
            pltpu.async_copy(w_hbm, w_v, sem),
            pltpu.async_copy(b_hbm, b_v, sem),
        ]
        for cp in cps:
            cp.wait()

        lane = lax.iota(jnp.int32, LANES)
        zero = jnp.zeros((LANES,), jnp.float32)
        zero_i = jnp.zeros((LANES,), jnp.int32)
        row = jnp.minimum(lane >> 2, jnp.full((LANES,), 2, jnp.int32))
        col = lane & jnp.full((LANES,), 3, jnp.int32)
        valid = lane < 12
        ep = jnp.where(valid, plsc.load_gather(e_v, [row, col]), zero)
        wp = jnp.where(valid, plsc.load_gather(w_v, [col, zero_i]), zero)
        bias = plsc.load_gather(b_v, [zero_i])

        scr_v[...] = ep * wp
        seg = lane & jnp.full((LANES,), -4, jnp.int32)
        one_i = jnp.ones((LANES,), jnp.int32)
        sum4 = (
            plsc.load_gather(scr_v, [seg])
            + plsc.load_gather(scr_v, [seg + one_i])
            + plsc.load_gather(scr_v, [seg + 2 * one_i])
            + plsc.load_gather(scr_v, [seg + 3 * one_i])
        )
        one = jnp.ones((LANES,), jnp.float32)
        sig = one / (one + jnp.exp(-(sum4 + bias)))
        scr_v[...] = sig
        table_idx = jnp.minimum(lane * 4, jnp.full((LANES,), 8, jnp.int32))
        scr_v[...] = plsc.load_gather(scr_v, [table_idx])

        for j in range(b_per_w // LANES):
            iv = idx_v[pl.ds(j * LANES, LANES)]
            out_v[pl.ds(j * LANES, LANES)] = plsc.load_gather(scr_v, [iv])

        pltpu.sync_copy(out_v, out_hbm.at[pl.ds(base, b_per_w)])

    return sc_kernel


def kernel(indices, emb_table, W, b):
    out = _build(indices.shape[0])(indices, emb_table, W, b)
    return out.reshape(-1, 1)


# speedup vs baseline: 2.2836x; 1.0014x over previous
import functools

import jax
import jax.numpy as jnp
from jax import lax
from jax.experimental import pallas as pl
from jax.experimental.pallas import tpu as pltpu
from jax.experimental.pallas import tpu_sc as plsc

LANES = 16


@functools.lru_cache(maxsize=None)
def _build(batch: int):
    info = plsc.get_sparse_core_info()
    nc, ns = info.num_cores, info.num_subcores
    nw = nc * ns
    assert batch % (8 * nw) == 0, "HBM 1-D slice offsets must be 8-aligned"
    b_per_w = batch // nw

    mesh = plsc.VectorSubcoreMesh(core_axis_name="c", subcore_axis_name="s")

    @functools.partial(
        pl.kernel,
        mesh=mesh,
        out_type=jax.ShapeDtypeStruct((batch,), jnp.float32),
        compiler_params=pltpu.CompilerParams(needs_layout_passes=False),
        scratch_types=[
            pltpu.VMEM((b_per_w,), jnp.int32),
            pltpu.VMEM((b_per_w,), jnp.float32),
            pltpu.VMEM((3, 4), jnp.float32),
            pltpu.VMEM((4, 1), jnp.float32),
            pltpu.VMEM((1,), jnp.float32),
            pltpu.VMEM((LANES,), jnp.float32),
            pltpu.SemaphoreType.DMA,
            pltpu.SemaphoreType.DMA,
        ],
    )
    def sc_kernel(idx_hbm, emb_hbm, w_hbm, b_hbm, out_hbm, idx_v, out_v, e_v, w_v, b_v, scr_v, sem_i, sem_w):
        wid = lax.axis_index("s") * nc + lax.axis_index("c")
        base = wid * b_per_w
        idx_cp = pltpu.async_copy(idx_hbm.at[pl.ds(base, b_per_w)], idx_v, sem_i)
        w_cps = [
            pltpu.async_copy(emb_hbm, e_v, sem_w),
            pltpu.async_copy(w_hbm, w_v, sem_w),
            pltpu.async_copy(b_hbm, b_v, sem_w),
        ]
        for cp in w_cps:
            cp.wait()

        lane = lax.iota(jnp.int32, LANES)
        zero = jnp.zeros((LANES,), jnp.float32)
        zero_i = jnp.zeros((LANES,), jnp.int32)
        row = jnp.minimum(lane >> 2, jnp.full((LANES,), 2, jnp.int32))
        col = lane & jnp.full((LANES,), 3, jnp.int32)
        valid = lane < 12
        ep = jnp.where(valid, plsc.load_gather(e_v, [row, col]), zero)
        wp = jnp.where(valid, plsc.load_gather(w_v, [col, zero_i]), zero)
        bias = plsc.load_gather(b_v, [zero_i])

        scr_v[...] = ep * wp
        seg = lane & jnp.full((LANES,), -4, jnp.int32)
        one_i = jnp.ones((LANES,), jnp.int32)
        sum4 = (
            plsc.load_gather(scr_v, [seg])
            + plsc.load_gather(scr_v, [seg + one_i])
            + plsc.load_gather(scr_v, [seg + 2 * one_i])
            + plsc.load_gather(scr_v, [seg + 3 * one_i])
        )
        one = jnp.ones((LANES,), jnp.float32)
        sig = one / (one + jnp.exp(-(sum4 + bias)))
        scr_v[...] = sig
        table_idx = jnp.minimum(lane * 4, jnp.full((LANES,), 8, jnp.int32))
        scr_v[...] = plsc.load_gather(scr_v, [table_idx])

        idx_cp.wait()
        for j in range(b_per_w // LANES):
            iv = idx_v[pl.ds(j * LANES, LANES)]
            out_v[pl.ds(j * LANES, LANES)] = plsc.load_gather(scr_v, [iv])

        pltpu.sync_copy(out_v, out_hbm.at[pl.ds(base, b_per_w)])

    return sc_kernel


def kernel(indices, emb_table, W, b):
    out = _build(indices.shape[0])(indices, emb_table, W, b)
    return out.reshape(-1, 1)


# speedup vs baseline: 2.3397x; 1.0246x over previous
import functools

import jax
import jax.numpy as jnp
from jax import lax
from jax.experimental import pallas as pl
from jax.experimental.pallas import tpu as pltpu
from jax.experimental.pallas import tpu_sc as plsc

LANES = 16


@functools.lru_cache(maxsize=None)
def _build(batch: int):
    info = plsc.get_sparse_core_info()
    nc, ns = info.num_cores, info.num_subcores
    nw = nc * ns
    assert batch % (8 * nw) == 0, "HBM 1-D slice offsets must be 8-aligned"
    b_per_w = batch // nw

    mesh = plsc.VectorSubcoreMesh(core_axis_name="c", subcore_axis_name="s")

    @functools.partial(
        pl.kernel,
        mesh=mesh,
        out_type=jax.ShapeDtypeStruct((batch,), jnp.float32),
        compiler_params=pltpu.CompilerParams(needs_layout_passes=False),
        scratch_types=[
            pltpu.VMEM((b_per_w,), jnp.int32),
            pltpu.VMEM((b_per_w,), jnp.float32),
            pltpu.VMEM((3, 4), jnp.float32),
            pltpu.VMEM((4, 1), jnp.float32),
            pltpu.VMEM((1,), jnp.float32),
            pltpu.VMEM((LANES,), jnp.float32),
            pltpu.SemaphoreType.DMA,
            pltpu.SemaphoreType.DMA,
        ],
    )
    def sc_kernel(idx_hbm, emb_hbm, w_hbm, b_hbm, out_hbm, idx_v, out_v, e_v, w_v, b_v, scr_v, sem_i, sem_w):
        wid = lax.axis_index("s") * nc + lax.axis_index("c")
        base = wid * b_per_w
        idx_cp = pltpu.async_copy(idx_hbm.at[pl.ds(base, b_per_w)], idx_v, sem_i)
        w_cps = [
            pltpu.async_copy(emb_hbm, e_v, sem_w),
            pltpu.async_copy(w_hbm, w_v, sem_w),
            pltpu.async_copy(b_hbm, b_v, sem_w),
        ]
        for cp in w_cps:
            cp.wait()

        lane = lax.iota(jnp.int32, LANES)
        zero = jnp.zeros((LANES,), jnp.float32)
        zero_i = jnp.zeros((LANES,), jnp.int32)
        row = jnp.minimum(lane >> 2, jnp.full((LANES,), 2, jnp.int32))
        col = lane & jnp.full((LANES,), 3, jnp.int32)
        valid = lane < 12
        ep = jnp.where(valid, plsc.load_gather(e_v, [row, col]), zero)
        wp = jnp.where(valid, plsc.load_gather(w_v, [col, zero_i]), zero)
        bias = plsc.load_gather(b_v, [zero_i])

        scr_v[...] = ep * wp
        seg = lane & jnp.full((LANES,), -4, jnp.int32)
        one_i = jnp.ones((LANES,), jnp.int32)
        sum4 = (
            plsc.load_gather(scr_v, [seg])
            + plsc.load_gather(scr_v, [seg + one_i])
            + plsc.load_gather(scr_v, [seg + 2 * one_i])
            + plsc.load_gather(scr_v, [seg + 3 * one_i])
        )
        one = jnp.ones((LANES,), jnp.float32)
        sig = one / (one + jnp.exp(-(sum4 + bias)))
        scr_v[...] = sig
        table_idx = jnp.minimum(lane * 4, jnp.full((LANES,), 8, jnp.int32))
        scr_v[...] = plsc.load_gather(scr_v, [table_idx])

        idx_cp.wait()

        def body(j, carry):
            o = j * LANES
            iv = idx_v[pl.ds(o, LANES)]
            out_v[pl.ds(o, LANES)] = plsc.load_gather(scr_v, [iv])
            return carry

        lax.fori_loop(0, b_per_w // LANES, body, 0)

        pltpu.sync_copy(out_v, out_hbm.at[pl.ds(base, b_per_w)])

    return sc_kernel


def kernel(indices, emb_table, W, b):
    out = _build(indices.shape[0])(indices, emb_table, W, b)
    return out.reshape(-1, 1)


# speedup vs baseline: 2.4996x; 1.0683x over previous
import functools

import jax
import jax.numpy as jnp
from jax import lax
from jax.experimental import pallas as pl
from jax.experimental.pallas import tpu as pltpu
from jax.experimental.pallas import tpu_sc as plsc

LANES = 16


@functools.lru_cache(maxsize=None)
def _build(batch: int):
    info = plsc.get_sparse_core_info()
    nc, ns = 1, info.num_subcores
    nw = nc * ns
    assert batch % (8 * nw) == 0, "HBM 1-D slice offsets must be 8-aligned"
    b_per_w = batch // nw

    mesh = plsc.VectorSubcoreMesh(
        core_axis_name="c", subcore_axis_name="s", num_cores=nc
    )

    @functools.partial(
        pl.kernel,
        mesh=mesh,
        out_type=jax.ShapeDtypeStruct((batch,), jnp.float32),
        compiler_params=pltpu.CompilerParams(needs_layout_passes=False),
        scratch_types=[
            pltpu.VMEM((b_per_w,), jnp.int32),
            pltpu.VMEM((b_per_w,), jnp.float32),
            pltpu.VMEM((3, 4), jnp.float32),
            pltpu.VMEM((4, 1), jnp.float32),
            pltpu.VMEM((1,), jnp.float32),
            pltpu.VMEM((LANES,), jnp.float32),
            pltpu.SemaphoreType.DMA,
            pltpu.SemaphoreType.DMA,
        ],
    )
    def sc_kernel(idx_hbm, emb_hbm, w_hbm, b_hbm, out_hbm, idx_v, out_v, e_v, w_v, b_v, scr_v, sem_i, sem_w):
        wid = lax.axis_index("s") * nc + lax.axis_index("c")
        base = wid * b_per_w
        idx_cp = pltpu.async_copy(idx_hbm.at[pl.ds(base, b_per_w)], idx_v, sem_i)
        w_cps = [
            pltpu.async_copy(emb_hbm, e_v, sem_w),
            pltpu.async_copy(w_hbm, w_v, sem_w),
            pltpu.async_copy(b_hbm, b_v, sem_w),
        ]
        for cp in w_cps:
            cp.wait()

        lane = lax.iota(jnp.int32, LANES)
        zero = jnp.zeros((LANES,), jnp.float32)
        zero_i = jnp.zeros((LANES,), jnp.int32)
        row = jnp.minimum(lane >> 2, jnp.full((LANES,), 2, jnp.int32))
        col = lane & jnp.full((LANES,), 3, jnp.int32)
        valid = lane < 12
        ep = jnp.where(valid, plsc.load_gather(e_v, [row, col]), zero)
        wp = jnp.where(valid, plsc.load_gather(w_v, [col, zero_i]), zero)
        bias = plsc.load_gather(b_v, [zero_i])

        scr_v[...] = ep * wp
        seg = lane & jnp.full((LANES,), -4, jnp.int32)
        one_i = jnp.ones((LANES,), jnp.int32)
        sum4 = (
            plsc.load_gather(scr_v, [seg])
            + plsc.load_gather(scr_v, [seg + one_i])
            + plsc.load_gather(scr_v, [seg + 2 * one_i])
            + plsc.load_gather(scr_v, [seg + 3 * one_i])
        )
        one = jnp.ones((LANES,), jnp.float32)
        sig = one / (one + jnp.exp(-(sum4 + bias)))
        scr_v[...] = sig
        table_idx = jnp.minimum(lane * 4, jnp.full((LANES,), 8, jnp.int32))
        scr_v[...] = plsc.load_gather(scr_v, [table_idx])

        idx_cp.wait()

        def body(j, carry):
            o = j * LANES
            iv = idx_v[pl.ds(o, LANES)]
            out_v[pl.ds(o, LANES)] = plsc.load_gather(scr_v, [iv])
            return carry

        lax.fori_loop(0, b_per_w // LANES, body, 0)

        pltpu.sync_copy(out_v, out_hbm.at[pl.ds(base, b_per_w)])

    return sc_kernel


def kernel(indices, emb_table, W, b):
    out = _build(indices.shape[0])(indices, emb_table, W, b)
    return out.reshape(-1, 1)
